# baseline (speedup 1.0000x reference)
import functools

import jax
import jax.numpy as jnp
from jax import lax
from jax.experimental import pallas as pl
from jax.experimental.pallas import tpu as pltpu
from jax.experimental.pallas import tpu_sc as plsc

VOCAB = 100000
D_MODEL = 1024
BATCH = 4
SEQ = 2048

_INFO = plsc.get_sparse_core_info()
_NC = _INFO.num_cores
_NS = _INFO.num_subcores
_NW = _NC * _NS
_SPW = SEQ // _NW
_C = 16
_NH = _SPW // _C
_NCHUNK = _NH * BATCH
_NBUF = 3
_L = 16
_DSL = D_MODEL // _L


def _emb_body(x_hbm, tab_hbm, pe_hbm, out_hbm,
              idx_v, buf0, buf1, buf2, pbuf,
              g0, g1, g2, s0, s1, s2, psem, isem):
    wid = lax.axis_index("s") * _NC + lax.axis_index("c")
    s_base = wid * _SPW
    bufs = (buf0, buf1, buf2)
    gsems = (g0, g1, g2)
    ssems = (s0, s1, s2)

    pe_cp = pltpu.async_copy(pe_hbm.at[pl.ds(s_base, _SPW)], pbuf, psem)
    idx_copies = [
        pltpu.async_copy(x_hbm.at[pl.ds(b * SEQ + s_base, _SPW)],
                         idx_v.at[pl.ds(b * _SPW, _SPW)], isem)
        for b in range(BATCH)
    ]
    for cp in idx_copies:
        cp.wait()

    def chunk_hb(i):
        return i // BATCH, i % BATCH

    def issue_gather(i):
        h, b = chunk_hb(i)
        return pltpu.async_copy(
            tab_hbm.at[idx_v.at[pl.ds(b * _SPW + h * _C, _C)]],
            bufs[i % _NBUF], gsems[i % _NBUF])

    def issue_store(i):
        h, b = chunk_hb(i)
        return pltpu.async_copy(
            bufs[i % _NBUF],
            out_hbm.at[pl.ds(b * SEQ + s_base + h * _C, _C)],
            ssems[i % _NBUF])

    gathers = [None] * _NCHUNK
    stores = [None] * _NCHUNK
    for i in range(_NBUF - 1):
        gathers[i] = issue_gather(i)
    pe_cp.wait()

    for i in range(_NCHUNK):
        nxt = i + _NBUF - 1
        if nxt < _NCHUNK:
            if nxt >= _NBUF:
                stores[nxt - _NBUF].wait()
            gathers[nxt] = issue_gather(nxt)
        gathers[i].wait()
        buf = bufs[i % _NBUF]
        h, _ = chunk_hb(i)

        def row_add(r, _):
            for k in range(_DSL):
                sl = pl.ds(k * _L, _L)
                buf[r, sl] = buf[r, sl] + pbuf[h * _C + r, sl]
            return ()

        lax.fori_loop(0, _C, row_add, ())
        stores[i] = issue_store(i)
    for i in range(_NCHUNK - _NBUF, _NCHUNK):
        stores[i].wait()


@jax.jit
def _emb(x_flat, tok_table, pe):
    mesh = plsc.VectorSubcoreMesh(core_axis_name="c", subcore_axis_name="s")
    k = pl.kernel(
        _emb_body,
        out_type=jax.ShapeDtypeStruct((BATCH * SEQ, D_MODEL), jnp.float32),
        mesh=mesh,
        scratch_types=[
            pltpu.VMEM((BATCH * _SPW,), jnp.int32),
            pltpu.VMEM((_C, D_MODEL), jnp.float32),
            pltpu.VMEM((_C, D_MODEL), jnp.float32),
            pltpu.VMEM((_C, D_MODEL), jnp.float32),
            pltpu.VMEM((_SPW, D_MODEL), jnp.float32),
            pltpu.SemaphoreType.DMA,
            pltpu.SemaphoreType.DMA,
            pltpu.SemaphoreType.DMA,
            pltpu.SemaphoreType.DMA,
            pltpu.SemaphoreType.DMA,
            pltpu.SemaphoreType.DMA,
            pltpu.SemaphoreType.DMA,
            pltpu.SemaphoreType.DMA,
        ],
    )
    return k(x_flat, tok_table, pe)


def kernel(x, tok_table, pe):
    out = _emb(x.reshape(-1), tok_table, pe)
    return out.reshape(BATCH, SEQ, D_MODEL)


# speedup vs baseline: 1.2676x; 1.2676x over previous
import functools

import jax
import jax.numpy as jnp
from jax import lax
from jax.experimental import pallas as pl
from jax.experimental.pallas import tpu as pltpu
from jax.experimental.pallas import tpu_sc as plsc

VOCAB = 100000
D_MODEL = 1024
BATCH = 4
SEQ = 2048

_INFO = plsc.get_sparse_core_info()
_NC = _INFO.num_cores
_NS = _INFO.num_subcores
_NW = _NC * _NS
_SPW = SEQ // _NW
_C = 32
_NH = _SPW // _C
_NCHUNK = _NH * BATCH
_L = 16
_DSL = D_MODEL // _L


def _emb_body(x_hbm, tab_hbm, pe_hbm, out_hbm,
              idx_v, buf0, buf1, pbuf, gsem, ssem, isem):
    wid = lax.axis_index("s") * _NC + lax.axis_index("c")
    s0 = wid * _SPW
    bufs = (buf0, buf1)

    idx_copies = [
        pltpu.async_copy(x_hbm.at[pl.ds(b * SEQ + s0, _SPW)],
                         idx_v.at[pl.ds(b * _SPW, _SPW)], isem)
        for b in range(BATCH)
    ]
    for cp in idx_copies:
        cp.wait()

    def chunk_hb(i):
        return i // BATCH, i % BATCH

    def issue_gather(i):
        h, b = chunk_hb(i)
        return pltpu.async_copy(
            tab_hbm.at[idx_v.at[pl.ds(b * _SPW + h * _C, _C)]],
            bufs[i % 2], gsem)

    def issue_store(i):
        h, b = chunk_hb(i)
        return pltpu.async_copy(
            bufs[i % 2],
            out_hbm.at[pl.ds(b * SEQ + s0 + h * _C, _C)], ssem)

    stores = [None] * _NCHUNK
    g_next = issue_gather(0)
    for i in range(_NCHUNK):
        h, b = chunk_hb(i)
        if b == 0:
            pltpu.sync_copy(pe_hbm.at[pl.ds(s0 + h * _C, _C)], pbuf)
        g_cur = g_next
        if i + 1 < _NCHUNK:
            if i >= 1:
                stores[i - 1].wait()
            g_next = issue_gather(i + 1)
        g_cur.wait()

        buf = bufs[i % 2]

        def row_add(r, _):
            for k in range(_DSL):
                sl = pl.ds(k * _L, _L)
                buf[r, sl] = buf[r, sl] + pbuf[r, sl]
            return ()

        lax.fori_loop(0, _C, row_add, ())
        stores[i] = issue_store(i)
    stores[_NCHUNK - 2].wait()
    stores[_NCHUNK - 1].wait()


@jax.jit
def _emb(x_flat, tok_table, pe):
    mesh = plsc.VectorSubcoreMesh(core_axis_name="c", subcore_axis_name="s")
    k = pl.kernel(
        _emb_body,
        out_type=jax.ShapeDtypeStruct((BATCH * SEQ, D_MODEL), jnp.float32),
        mesh=mesh,
        scratch_types=[
            pltpu.VMEM((BATCH * _SPW,), jnp.int32),
            pltpu.VMEM((_C, D_MODEL), jnp.float32),
            pltpu.VMEM((_C, D_MODEL), jnp.float32),
            pltpu.VMEM((_C, D_MODEL), jnp.float32),
            pltpu.SemaphoreType.DMA,
            pltpu.SemaphoreType.DMA,
            pltpu.SemaphoreType.DMA,
        ],
    )
    return k(x_flat, tok_table, pe)


def kernel(x, tok_table, pe):
    out = _emb(x.reshape(-1), tok_table, pe)
    return out.reshape(BATCH, SEQ, D_MODEL)


# speedup vs baseline: 1.2681x; 1.0004x over previous
import functools

import jax
import jax.numpy as jnp
from jax import lax
from jax.experimental import pallas as pl
from jax.experimental.pallas import tpu as pltpu
from jax.experimental.pallas import tpu_sc as plsc

VOCAB = 100000
D_MODEL = 1024
BATCH = 4
SEQ = 2048

_INFO = plsc.get_sparse_core_info()
_NC = _INFO.num_cores
_NS = _INFO.num_subcores
_NW = _NC * _NS
_SPW = SEQ // _NW
_C = 32
_NH = _SPW // _C
_NCHUNK = _NH * BATCH
_L = 16
_DSL = D_MODEL // _L


def _emb_body(x_hbm, tab_hbm, pe_hbm, out_hbm,
              idx_v, buf0, buf1, pbuf, gsem, ssem, isem, psem):
    wid = lax.axis_index("s") * _NC + lax.axis_index("c")
    s0 = wid * _SPW
    bufs = (buf0, buf1)

    pltpu.sync_copy(x_hbm.at[pl.ds(wid * BATCH * _SPW, BATCH * _SPW)], idx_v)

    def chunk_hb(i):
        return i // BATCH, i % BATCH

    def issue_gather(i):
        h, b = chunk_hb(i)
        return pltpu.async_copy(
            tab_hbm.at[idx_v.at[pl.ds(b * _SPW + h * _C, _C)]],
            bufs[i % 2], gsem)

    def issue_store(i):
        h, b = chunk_hb(i)
        return pltpu.async_copy(
            bufs[i % 2],
            out_hbm.at[pl.ds(b * SEQ + s0 + h * _C, _C)], ssem)

    stores = [None] * _NCHUNK
    pe_cp = pltpu.async_copy(pe_hbm.at[pl.ds(s0, _C)], pbuf, psem)
    g_next = issue_gather(0)
    for i in range(_NCHUNK):
        h, b = chunk_hb(i)
        if b == 0 and h > 0:
            pltpu.sync_copy(pe_hbm.at[pl.ds(s0 + h * _C, _C)], pbuf)
        g_cur = g_next
        if i + 1 < _NCHUNK:
            if i >= 1:
                stores[i - 1].wait()
            g_next = issue_gather(i + 1)
        g_cur.wait()
        if i == 0:
            pe_cp.wait()

        buf = bufs[i % 2]

        def row_add(r, _):
            for k in range(_DSL):
                sl = pl.ds(k * _L, _L)
                buf[r, sl] = buf[r, sl] + pbuf[r, sl]
            return ()

        lax.fori_loop(0, _C, row_add, ())
        stores[i] = issue_store(i)
    stores[_NCHUNK - 2].wait()
    stores[_NCHUNK - 1].wait()


@jax.jit
def _emb(x_flat, tok_table, pe):
    mesh = plsc.VectorSubcoreMesh(core_axis_name="c", subcore_axis_name="s")
    k = pl.kernel(
        _emb_body,
        out_type=jax.ShapeDtypeStruct((BATCH * SEQ, D_MODEL), jnp.float32),
        mesh=mesh,
        scratch_types=[
            pltpu.VMEM((BATCH * _SPW,), jnp.int32),
            pltpu.VMEM((_C, D_MODEL), jnp.float32),
            pltpu.VMEM((_C, D_MODEL), jnp.float32),
            pltpu.VMEM((_C, D_MODEL), jnp.float32),
            pltpu.SemaphoreType.DMA,
            pltpu.SemaphoreType.DMA,
            pltpu.SemaphoreType.DMA,
            pltpu.SemaphoreType.DMA,
        ],
    )
    return k(x_flat, tok_table, pe)


def kernel(x, tok_table, pe):
    x_perm = x.reshape(BATCH, _NW, _SPW).transpose(1, 0, 2).reshape(-1)
    out = _emb(x_perm, tok_table, pe)
    return out.reshape(BATCH, SEQ, D_MODEL)
